# baseline (device time: 273236 ns/iter reference)
import jax
import jax.numpy as jnp
from jax import lax
from jax.experimental import pallas as pl
from jax.experimental.pallas import tpu as pltpu

S = 2048
S_HALF = 1024
K = 4096
N = 8192
TILE_N = 512
NT = N // TILE_N

WIRE_DTYPE = jnp.bfloat16


def kernel(O, Wo):
    O2 = O.reshape(S, K).astype(jnp.bfloat16)

    def body(o_ref, w_ref, out_ref,
             wb_buf, send_buf, recv_buf, mine_buf, send_sems, recv_sems):
        j = pl.program_id(0)
        my_y = lax.axis_index("y")
        peer = (lax.axis_index("x"), 1 - my_y, lax.axis_index("z"))
        slot = j % 2
        pslot = (j + 1) % 2

        my_off = my_y * S_HALF
        peer_off = (1 - my_y) * S_HALF

        def make_rdma(s):
            return pltpu.make_async_remote_copy(
                src_ref=send_buf.at[s],
                dst_ref=recv_buf.at[s],
                send_sem=send_sems.at[s],
                recv_sem=recv_sems.at[s],
                device_id=peer,
                device_id_type=pl.DeviceIdType.MESH,
            )

        @pl.when(j == 0)
        def _():
            barrier_sem = pltpu.get_barrier_semaphore()
            pl.semaphore_signal(
                barrier_sem, inc=1,
                device_id=peer, device_id_type=pl.DeviceIdType.MESH,
            )
            pl.semaphore_wait(barrier_sem, 1)

        @pl.when(jnp.logical_and(j >= 2, j < NT))
        def _():
            make_rdma(slot).wait_send()

        @pl.when(j < NT)
        def _():
            wb_buf[:, :] = w_ref[:, :].astype(jnp.bfloat16)
            theirs = jnp.dot(
                o_ref[pl.ds(peer_off, S_HALF), :], wb_buf[:, :],
                preferred_element_type=jnp.float32,
            )
            send_buf[slot, :, :] = theirs.astype(WIRE_DTYPE)

        @pl.when(j >= 1)
        def _():
            make_rdma(pslot).wait_recv()

        @pl.when(j >= 1)
        def _():
            out_ref[0, :, :] = (
                mine_buf[pslot, :, :]
                + recv_buf[pslot, :, :].astype(jnp.float32)
            )

        @pl.when(j < NT)
        def _():
            make_rdma(slot).start()

        @pl.when(j < NT)
        def _():
            mine_buf[slot, :, :] = jnp.dot(
                o_ref[pl.ds(my_off, S_HALF), :], wb_buf[:, :],
                preferred_element_type=jnp.float32,
            )

        @pl.when(j == NT)
        def _():
            make_rdma((NT - 2) % 2).wait_send()
            make_rdma((NT - 1) % 2).wait_send()

    return pl.pallas_call(
        body,
        grid=(NT + 1,),
        out_shape=jax.ShapeDtypeStruct((1, S_HALF, N), jnp.float32),
        in_specs=[
            pl.BlockSpec((S, K), lambda j: (0, 0)),
            pl.BlockSpec((K, TILE_N), lambda j: (0, jnp.minimum(j, NT - 1))),
        ],
        out_specs=pl.BlockSpec(
            (1, S_HALF, TILE_N), lambda j: (0, 0, jnp.maximum(j - 1, 0))
        ),
        scratch_shapes=[
            pltpu.VMEM((K, TILE_N), jnp.bfloat16),
            pltpu.VMEM((2, S_HALF, TILE_N), WIRE_DTYPE),
            pltpu.VMEM((2, S_HALF, TILE_N), WIRE_DTYPE),
            pltpu.VMEM((2, S_HALF, TILE_N), jnp.float32),
            pltpu.SemaphoreType.DMA((2,)),
            pltpu.SemaphoreType.DMA((2,)),
        ],
        compiler_params=pltpu.CompilerParams(
            collective_id=0,
            dimension_semantics=("arbitrary",),
            vmem_limit_bytes=60 * 1024 * 1024,
        ),
    )(O2, Wo)


# device time: 271201 ns/iter; 1.0075x vs baseline; 1.0075x over previous
import jax
import jax.numpy as jnp
from jax import lax
from jax.experimental import pallas as pl
from jax.experimental.pallas import tpu as pltpu

S = 2048
S_HALF = 1024
K = 4096
N = 8192
TILE_N = 512
NT = N // TILE_N

WIRE_DTYPE = jnp.bfloat16


def kernel(O, Wo):
    O2 = O.reshape(S, K).astype(jnp.bfloat16)

    def body(o_ref, w_ref, out_ref,
             wb_buf, send_buf, recv_buf, mine_buf, send_sems, recv_sems):
        j = pl.program_id(0)
        my_y = lax.axis_index("y")
        peer = (lax.axis_index("x"), 1 - my_y, lax.axis_index("z"))
        slot = j % 2
        pslot = (j + 1) % 2

        my_off = my_y * S_HALF
        peer_off = (1 - my_y) * S_HALF

        def make_rdma(s):
            return pltpu.make_async_remote_copy(
                src_ref=send_buf.at[s],
                dst_ref=recv_buf.at[s],
                send_sem=send_sems.at[s],
                recv_sem=recv_sems.at[s],
                device_id=peer,
                device_id_type=pl.DeviceIdType.MESH,
            )

        @pl.when(j == 0)
        def _():
            barrier_sem = pltpu.get_barrier_semaphore()
            pl.semaphore_signal(
                barrier_sem, inc=1,
                device_id=peer, device_id_type=pl.DeviceIdType.MESH,
            )
            pl.semaphore_wait(barrier_sem, 1)

        @pl.when(jnp.logical_and(j >= 2, j < NT))
        def _():
            make_rdma(slot).wait_send()

        @pl.when(j < NT)
        def _():
            wb_buf[:, :] = w_ref[:, :].astype(jnp.bfloat16)
            send_buf[slot, :, :] = jnp.dot(
                o_ref[pl.ds(peer_off, S_HALF), :], wb_buf[:, :],
                preferred_element_type=jnp.float32,
            ).astype(WIRE_DTYPE)

        @pl.when(j >= 1)
        def _():
            make_rdma(pslot).wait_recv()

        @pl.when(j >= 1)
        def _():
            out_ref[0, :, :] = (
                mine_buf[pslot, :, :].astype(jnp.float32)
                + recv_buf[pslot, :, :].astype(jnp.float32)
            )

        @pl.when(j < NT)
        def _():
            make_rdma(slot).start()

        @pl.when(j < NT)
        def _():
            mine_buf[slot, :, :] = jnp.dot(
                o_ref[pl.ds(my_off, S_HALF), :], wb_buf[:, :],
                preferred_element_type=jnp.float32,
            ).astype(jnp.bfloat16)

        @pl.when(j == NT)
        def _():
            make_rdma((NT - 2) % 2).wait_send()
            make_rdma((NT - 1) % 2).wait_send()

    return pl.pallas_call(
        body,
        grid=(NT + 1,),
        out_shape=jax.ShapeDtypeStruct((1, S_HALF, N), jnp.float32),
        in_specs=[
            pl.BlockSpec((S, K), lambda j: (0, 0)),
            pl.BlockSpec((K, TILE_N), lambda j: (0, jnp.minimum(j, NT - 1))),
        ],
        out_specs=pl.BlockSpec(
            (1, S_HALF, TILE_N), lambda j: (0, 0, jnp.maximum(j - 1, 0))
        ),
        scratch_shapes=[
            pltpu.VMEM((K, TILE_N), jnp.bfloat16),
            pltpu.VMEM((2, S_HALF, TILE_N), WIRE_DTYPE),
            pltpu.VMEM((2, S_HALF, TILE_N), WIRE_DTYPE),
            pltpu.VMEM((2, S_HALF, TILE_N), jnp.bfloat16),
            pltpu.SemaphoreType.DMA((2,)),
            pltpu.SemaphoreType.DMA((2,)),
        ],
        compiler_params=pltpu.CompilerParams(
            collective_id=0,
            dimension_semantics=("arbitrary",),
            vmem_limit_bytes=60 * 1024 * 1024,
        ),
    )(O2, Wo)


# device time: 247374 ns/iter; 1.1045x vs baseline; 1.0963x over previous
import jax
import jax.numpy as jnp
from jax import lax
from jax.experimental import pallas as pl
from jax.experimental.pallas import tpu as pltpu

S = 2048
S_HALF = 1024
K = 4096
N = 8192
TILE_N = 512
NT = N // TILE_N

WIRE_DTYPE = jnp.bfloat16


def kernel(O, Wo):
    O2 = O.reshape(S, K).astype(jnp.bfloat16)

    def body(o_ref, w_ref, out_ref,
             wb_buf, send_buf, recv_buf, mine_buf, send_sems, recv_sems):
        j = pl.program_id(0)
        my_y = lax.axis_index("y")
        peer = (lax.axis_index("x"), 1 - my_y, lax.axis_index("z"))
        slot = j % 4
        pslot = (j + 3) % 4
        mslot = j % 2
        pmslot = (j + 1) % 2

        my_off = my_y * S_HALF
        peer_off = (1 - my_y) * S_HALF

        def make_rdma(s):
            return pltpu.make_async_remote_copy(
                src_ref=send_buf.at[s],
                dst_ref=recv_buf.at[s],
                send_sem=send_sems.at[s],
                recv_sem=recv_sems.at[s],
                device_id=peer,
                device_id_type=pl.DeviceIdType.MESH,
            )

        @pl.when(j == 0)
        def _():
            barrier_sem = pltpu.get_barrier_semaphore()
            pl.semaphore_signal(
                barrier_sem, inc=1,
                device_id=peer, device_id_type=pl.DeviceIdType.MESH,
            )
            pl.semaphore_wait(barrier_sem, 1)

        @pl.when(jnp.logical_and(j >= 4, j < NT))
        def _():
            make_rdma(slot).wait_send()

        @pl.when(j < NT)
        def _():
            wb_buf[:, :] = w_ref[:, :].astype(jnp.bfloat16)
            send_buf[slot, :, :] = jnp.dot(
                o_ref[pl.ds(peer_off, S_HALF), :], wb_buf[:, :],
                preferred_element_type=jnp.float32,
            ).astype(WIRE_DTYPE)

        @pl.when(j < NT)
        def _():
            make_rdma(slot).start()

        @pl.when(j >= 1)
        def _():
            make_rdma(pslot).wait_recv()
            out_ref[0, :, :] = (
                mine_buf[pmslot, :, :].astype(jnp.float32)
                + recv_buf[pslot, :, :].astype(jnp.float32)
            )

        @pl.when(j < NT)
        def _():
            mine_buf[mslot, :, :] = jnp.dot(
                o_ref[pl.ds(my_off, S_HALF), :], wb_buf[:, :],
                preferred_element_type=jnp.float32,
            ).astype(jnp.bfloat16)

        @pl.when(j == NT)
        def _():
            for s in range(4):
                make_rdma(s).wait_send()

    return pl.pallas_call(
        body,
        grid=(NT + 1,),
        out_shape=jax.ShapeDtypeStruct((1, S_HALF, N), jnp.float32),
        in_specs=[
            pl.BlockSpec((S, K), lambda j: (0, 0)),
            pl.BlockSpec((K, TILE_N), lambda j: (0, jnp.minimum(j, NT - 1))),
        ],
        out_specs=pl.BlockSpec(
            (1, S_HALF, TILE_N), lambda j: (0, 0, jnp.maximum(j - 1, 0))
        ),
        scratch_shapes=[
            pltpu.VMEM((K, TILE_N), jnp.bfloat16),
            pltpu.VMEM((4, S_HALF, TILE_N), WIRE_DTYPE),
            pltpu.VMEM((4, S_HALF, TILE_N), WIRE_DTYPE),
            pltpu.VMEM((2, S_HALF, TILE_N), jnp.bfloat16),
            pltpu.SemaphoreType.DMA((4,)),
            pltpu.SemaphoreType.DMA((4,)),
        ],
        compiler_params=pltpu.CompilerParams(
            collective_id=0,
            dimension_semantics=("arbitrary",),
            vmem_limit_bytes=60 * 1024 * 1024,
        ),
    )(O2, Wo)


# device time: 169648 ns/iter; 1.6106x vs baseline; 1.4582x over previous
import jax
import jax.numpy as jnp
from jax import lax
from jax.experimental import pallas as pl
from jax.experimental.pallas import tpu as pltpu

S = 2048
S_HALF = 1024
K = 4096
N = 8192
HALF_N = N // 2
TILE_N = 256
NT = HALF_N // TILE_N


def kernel(O, Wo):
    O2 = O.reshape(S, K).astype(jnp.bfloat16)
    xi = jnp.reshape(lax.axis_index("x").astype(jnp.int32), (1,))

    def body(xi_ref, o_ref, w_ref, out_ref,
             wb_buf, ysend, yrecv, mine_buf, sum_buf, xsend, xrecv, pr_buf,
             ysend_sems, yrecv_sems, xsend_sems, xrecv_sems,
             sumdma_sems, prdma_sems):
        j = pl.program_id(0)
        my_x = lax.axis_index("x")
        my_y = lax.axis_index("y")
        my_z = lax.axis_index("z")
        y_peer = (my_x, 1 - my_y, my_z)
        x_peer = (1 - my_x, my_y, my_z)

        my_off = my_y * S_HALF
        peer_off = (1 - my_y) * S_HALF
        my_base = my_x * HALF_N
        pr_base = (1 - my_x) * HALF_N

        def yrdma(s):
            return pltpu.make_async_remote_copy(
                src_ref=ysend.at[s], dst_ref=yrecv.at[s],
                send_sem=ysend_sems.at[s], recv_sem=yrecv_sems.at[s],
                device_id=y_peer, device_id_type=pl.DeviceIdType.MESH,
            )

        def xrdma(s):
            return pltpu.make_async_remote_copy(
                src_ref=xsend.at[s], dst_ref=xrecv.at[s],
                send_sem=xsend_sems.at[s], recv_sem=xrecv_sems.at[s],
                device_id=x_peer, device_id_type=pl.DeviceIdType.MESH,
            )

        def sumcopy(s2, col):
            return pltpu.make_async_copy(
                sum_buf.at[s2],
                out_ref.at[0, :, pl.ds(col, TILE_N)],
                sumdma_sems.at[s2],
            )

        def prcopy(s2, col):
            return pltpu.make_async_copy(
                pr_buf.at[s2],
                out_ref.at[0, :, pl.ds(col, TILE_N)],
                prdma_sems.at[s2],
            )

        @pl.when(j == 0)
        def _():
            barrier_sem = pltpu.get_barrier_semaphore()
            for nbr in (y_peer, x_peer):
                pl.semaphore_signal(
                    barrier_sem, inc=1,
                    device_id=nbr, device_id_type=pl.DeviceIdType.MESH,
                )
            pl.semaphore_wait(barrier_sem, 2)

        @pl.when(jnp.logical_and(j >= 4, j < NT))
        def _():
            yrdma(j % 4).wait_send()

        @pl.when(j < NT)
        def _():
            wb_buf[:, :] = w_ref[:, :].astype(jnp.bfloat16)
            ysend[j % 4, :, :] = jnp.dot(
                o_ref[pl.ds(peer_off, S_HALF), :], wb_buf[:, :],
                preferred_element_type=jnp.float32,
            ).astype(jnp.bfloat16)
            yrdma(j % 4).start()

        @pl.when(jnp.logical_and(j >= 1, j <= NT))
        def _():
            t = j - 1
            ts2 = t % 2
            ts4 = t % 4
            yrdma(ts4).wait_recv()

            @pl.when(t >= 2)
            def _():
                sumcopy(ts2, my_base + (t - 2) * TILE_N).wait()

            @pl.when(t >= 4)
            def _():
                xrdma(ts4).wait_send()

            s = (mine_buf[ts2, :, :].astype(jnp.float32)
                 + yrecv[ts4, :, :].astype(jnp.float32))
            sum_buf[ts2, :, :] = s
            xsend[ts4, :, :] = s.astype(jnp.bfloat16)
            xrdma(ts4).start()
            sumcopy(ts2, my_base + t * TILE_N).start()

        @pl.when(j < NT)
        def _():
            mine_buf[j % 2, :, :] = jnp.dot(
                o_ref[pl.ds(my_off, S_HALF), :], wb_buf[:, :],
                preferred_element_type=jnp.float32,
            ).astype(jnp.bfloat16)

        @pl.when(j >= 2)
        def _():
            t = j - 2
            ts2 = t % 2
            ts4 = t % 4
            xrdma(ts4).wait_recv()

            @pl.when(t >= 2)
            def _():
                prcopy(ts2, pr_base + (t - 2) * TILE_N).wait()

            pr_buf[ts2, :, :] = xrecv[ts4, :, :].astype(jnp.float32)
            prcopy(ts2, pr_base + t * TILE_N).start()

        @pl.when(j == NT + 1)
        def _():
            for s in range(4):
                yrdma(s).wait_send()
                xrdma(s).wait_send()
            for t in (NT - 2, NT - 1):
                sumcopy(t % 2, my_base + t * TILE_N).wait()
                prcopy(t % 2, pr_base + t * TILE_N).wait()

    grid_spec = pltpu.PrefetchScalarGridSpec(
        num_scalar_prefetch=1,
        grid=(NT + 2,),
        in_specs=[
            pl.BlockSpec((S, K), lambda j, xi_ref: (0, 0)),
            pl.BlockSpec(
                (K, TILE_N),
                lambda j, xi_ref: (0, xi_ref[0] * NT + jnp.minimum(j, NT - 1)),
            ),
        ],
        out_specs=pl.BlockSpec(memory_space=pl.ANY),
        scratch_shapes=[
            pltpu.VMEM((K, TILE_N), jnp.bfloat16),
            pltpu.VMEM((4, S_HALF, TILE_N), jnp.bfloat16),
            pltpu.VMEM((4, S_HALF, TILE_N), jnp.bfloat16),
            pltpu.VMEM((2, S_HALF, TILE_N), jnp.bfloat16),
            pltpu.VMEM((2, S_HALF, TILE_N), jnp.float32),
            pltpu.VMEM((4, S_HALF, TILE_N), jnp.bfloat16),
            pltpu.VMEM((4, S_HALF, TILE_N), jnp.bfloat16),
            pltpu.VMEM((2, S_HALF, TILE_N), jnp.float32),
            pltpu.SemaphoreType.DMA((4,)),
            pltpu.SemaphoreType.DMA((4,)),
            pltpu.SemaphoreType.DMA((4,)),
            pltpu.SemaphoreType.DMA((4,)),
            pltpu.SemaphoreType.DMA((2,)),
            pltpu.SemaphoreType.DMA((2,)),
        ],
    )

    return pl.pallas_call(
        body,
        grid_spec=grid_spec,
        out_shape=jax.ShapeDtypeStruct((1, S_HALF, N), jnp.float32),
        compiler_params=pltpu.CompilerParams(
            collective_id=0,
            dimension_semantics=("arbitrary",),
            vmem_limit_bytes=60 * 1024 * 1024,
        ),
    )(xi, O2, Wo)


# device time: 156799 ns/iter; 1.7426x vs baseline; 1.0819x over previous
import jax
import jax.numpy as jnp
from jax import lax
from jax.experimental import pallas as pl
from jax.experimental.pallas import tpu as pltpu

S = 2048
S_HALF = 1024
S_QTR = 512
K = 4096
N = 8192
HALF_N = N // 2
TILE_N = 256
NT = HALF_N // TILE_N


def kernel(O, Wo):
    O2 = O.reshape(S, K).astype(jnp.bfloat16)
    xi = jnp.reshape(lax.axis_index("x").astype(jnp.int32), (1,))

    def body(xi_ref, o_ref, w_ref, out_ref,
             wb_buf, ysend, yrecv, mine_buf, sum_buf, sumbf,
             xrecv, zb_src, zrecva, zrecvb, xstore, zastore, zbstore,
             ysend_sems, yrecv_sems, xsend_sems, xrecv_sems,
             zasend_sems, zarecv_sems, zbsend_sems, zbrecv_sems,
             sumdma_sems, xdma_sems, zadma_sems, zbdma_sems):
        j = pl.program_id(0)
        my_x = lax.axis_index("x")
        my_y = lax.axis_index("y")
        my_z = lax.axis_index("z")
        zp = my_z % 2
        y_peer = (my_x, 1 - my_y, my_z)
        x_peer = (1 - my_x, my_y, my_z)
        z_twin = (my_x, my_y, my_z + 1 - 2 * zp)

        my_off = my_y * S_HALF
        peer_off = (1 - my_y) * S_HALF
        my_base = my_x * HALF_N
        pr_base = (1 - my_x) * HALF_N
        my_rows = zp * S_QTR
        tw_rows = (1 - zp) * S_QTR

        def yrdma(s):
            return pltpu.make_async_remote_copy(
                src_ref=ysend.at[s], dst_ref=yrecv.at[s],
                send_sem=ysend_sems.at[s], recv_sem=yrecv_sems.at[s],
                device_id=y_peer, device_id_type=pl.DeviceIdType.MESH,
            )

        def xrdma(s):
            return pltpu.make_async_remote_copy(
                src_ref=sumbf.at[s], dst_ref=xrecv.at[s],
                send_sem=xsend_sems.at[s], recv_sem=xrecv_sems.at[s],
                device_id=x_peer, device_id_type=pl.DeviceIdType.MESH,
            )

        def zardma(s):
            return pltpu.make_async_remote_copy(
                src_ref=sumbf.at[s], dst_ref=zrecva.at[s],
                send_sem=zasend_sems.at[s], recv_sem=zarecv_sems.at[s],
                device_id=z_twin, device_id_type=pl.DeviceIdType.MESH,
            )

        def zbrdma(s, s2):
            return pltpu.make_async_remote_copy(
                src_ref=zb_src.at[s2], dst_ref=zrecvb.at[s],
                send_sem=zbsend_sems.at[s], recv_sem=zbrecv_sems.at[s],
                device_id=z_twin, device_id_type=pl.DeviceIdType.MESH,
            )

        def outcopy(src, s2, sems, row, col):
            return pltpu.make_async_copy(
                src.at[s2],
                out_ref.at[0, pl.ds(row, S_QTR), pl.ds(col, TILE_N)],
                sems.at[s2],
            )

        @pl.when(j == 0)
        def _():
            barrier_sem = pltpu.get_barrier_semaphore()
            for nbr in (y_peer, x_peer, z_twin):
                pl.semaphore_signal(
                    barrier_sem, inc=1,
                    device_id=nbr, device_id_type=pl.DeviceIdType.MESH,
                )
            pl.semaphore_wait(barrier_sem, 3)

        @pl.when(jnp.logical_and(j >= 4, j < NT))
        def _():
            yrdma(j % 4).wait_send()

        @pl.when(j < NT)
        def _():
            wb_buf[:, :] = w_ref[:, :].astype(jnp.bfloat16)
            ysend[j % 4, :, :] = jnp.dot(
                o_ref[pl.ds(peer_off + zp * S_QTR, S_QTR), :], wb_buf[:, :],
                preferred_element_type=jnp.float32,
            ).astype(jnp.bfloat16)
            yrdma(j % 4).start()

        @pl.when(jnp.logical_and(j >= 1, j <= NT))
        def _():
            t = j - 1
            ts2 = t % 2
            ts4 = t % 4
            yrdma(ts4).wait_recv()

            @pl.when(t >= 2)
            def _():
                outcopy(sum_buf, ts2, sumdma_sems,
                        my_rows, my_base + (t - 2) * TILE_N).wait()

            @pl.when(t >= 4)
            def _():
                xrdma(ts4).wait_send()
                zardma(ts4).wait_send()

            s = (mine_buf[ts2, :, :].astype(jnp.float32)
                 + yrecv[ts4, :, :].astype(jnp.float32))
            sum_buf[ts2, :, :] = s
            sumbf[ts4, :, :] = s.astype(jnp.bfloat16)
            xrdma(ts4).start()
            zardma(ts4).start()
            outcopy(sum_buf, ts2, sumdma_sems,
                    my_rows, my_base + t * TILE_N).start()

        @pl.when(j < NT)
        def _():
            mine_buf[j % 2, :, :] = jnp.dot(
                o_ref[pl.ds(my_off + zp * S_QTR, S_QTR), :], wb_buf[:, :],
                preferred_element_type=jnp.float32,
            ).astype(jnp.bfloat16)

        @pl.when(jnp.logical_and(j >= 2, j <= NT + 1))
        def _():
            t = j - 2
            ts2 = t % 2
            ts4 = t % 4
            xrdma(ts4).wait_recv()

            @pl.when(t >= 2)
            def _():
                zbrdma((t - 2) % 4, ts2).wait_send()

            @pl.when(t >= 2)
            def _():
                outcopy(xstore, ts2, xdma_sems,
                        my_rows, pr_base + (t - 2) * TILE_N).wait()

            zb_src[ts2, :, :] = xrecv[ts4, :, :]
            zbrdma(ts4, ts2).start()
            xstore[ts2, :, :] = xrecv[ts4, :, :].astype(jnp.float32)
            outcopy(xstore, ts2, xdma_sems,
                    my_rows, pr_base + t * TILE_N).start()

        @pl.when(jnp.logical_and(j >= 2, j <= NT + 1))
        def _():
            t = j - 2
            ts2 = t % 2
            ts4 = t % 4
            zardma(ts4).wait_recv()

            @pl.when(t >= 2)
            def _():
                outcopy(zastore, ts2, zadma_sems,
                        tw_rows, my_base + (t - 2) * TILE_N).wait()

            zastore[ts2, :, :] = zrecva[ts4, :, :].astype(jnp.float32)
            outcopy(zastore, ts2, zadma_sems,
                    tw_rows, my_base + t * TILE_N).start()

        @pl.when(jnp.logical_and(j >= 3, j <= NT + 2))
        def _():
            t = j - 3
            ts2 = t % 2
            ts4 = t % 4
            zbrdma(ts4, ts2).wait_recv()

            @pl.when(t >= 2)
            def _():
                outcopy(zbstore, ts2, zbdma_sems,
                        tw_rows, pr_base + (t - 2) * TILE_N).wait()

            zbstore[ts2, :, :] = zrecvb[ts4, :, :].astype(jnp.float32)
            outcopy(zbstore, ts2, zbdma_sems,
                    tw_rows, pr_base + t * TILE_N).start()

        @pl.when(j == NT + 2)
        def _():
            for s in range(4):
                yrdma(s).wait_send()
                xrdma(s).wait_send()
                zardma(s).wait_send()
            for t in (NT - 2, NT - 1):
                zbrdma(t % 4, t % 2).wait_send()
                outcopy(sum_buf, t % 2, sumdma_sems,
                        my_rows, my_base + t * TILE_N).wait()
                outcopy(xstore, t % 2, xdma_sems,
                        my_rows, pr_base + t * TILE_N).wait()
                outcopy(zastore, t % 2, zadma_sems,
                        tw_rows, my_base + t * TILE_N).wait()
                outcopy(zbstore, t % 2, zbdma_sems,
                        tw_rows, pr_base + t * TILE_N).wait()

    grid_spec = pltpu.PrefetchScalarGridSpec(
        num_scalar_prefetch=1,
        grid=(NT + 3,),
        in_specs=[
            pl.BlockSpec((S, K), lambda j, xi_ref: (0, 0)),
            pl.BlockSpec(
                (K, TILE_N),
                lambda j, xi_ref: (0, xi_ref[0] * NT + jnp.minimum(j, NT - 1)),
            ),
        ],
        out_specs=pl.BlockSpec(memory_space=pl.ANY),
        scratch_shapes=[
            pltpu.VMEM((K, TILE_N), jnp.bfloat16),
            pltpu.VMEM((4, S_QTR, TILE_N), jnp.bfloat16),
            pltpu.VMEM((4, S_QTR, TILE_N), jnp.bfloat16),
            pltpu.VMEM((2, S_QTR, TILE_N), jnp.bfloat16),
            pltpu.VMEM((2, S_QTR, TILE_N), jnp.float32),
            pltpu.VMEM((4, S_QTR, TILE_N), jnp.bfloat16),
            pltpu.VMEM((4, S_QTR, TILE_N), jnp.bfloat16),
            pltpu.VMEM((2, S_QTR, TILE_N), jnp.bfloat16),
            pltpu.VMEM((4, S_QTR, TILE_N), jnp.bfloat16),
            pltpu.VMEM((4, S_QTR, TILE_N), jnp.bfloat16),
            pltpu.VMEM((2, S_QTR, TILE_N), jnp.float32),
            pltpu.VMEM((2, S_QTR, TILE_N), jnp.float32),
            pltpu.VMEM((2, S_QTR, TILE_N), jnp.float32),
            pltpu.SemaphoreType.DMA((4,)),
            pltpu.SemaphoreType.DMA((4,)),
            pltpu.SemaphoreType.DMA((4,)),
            pltpu.SemaphoreType.DMA((4,)),
            pltpu.SemaphoreType.DMA((4,)),
            pltpu.SemaphoreType.DMA((4,)),
            pltpu.SemaphoreType.DMA((4,)),
            pltpu.SemaphoreType.DMA((4,)),
            pltpu.SemaphoreType.DMA((2,)),
            pltpu.SemaphoreType.DMA((2,)),
            pltpu.SemaphoreType.DMA((2,)),
            pltpu.SemaphoreType.DMA((2,)),
        ],
    )

    return pl.pallas_call(
        body,
        grid_spec=grid_spec,
        out_shape=jax.ShapeDtypeStruct((1, S_HALF, N), jnp.float32),
        compiler_params=pltpu.CompilerParams(
            collective_id=0,
            dimension_semantics=("arbitrary",),
            vmem_limit_bytes=60 * 1024 * 1024,
        ),
    )(xi, O2, Wo)
